# Initial kernel scaffold; baseline (speedup 1.0000x reference)
import jax
import jax.numpy as jnp
from jax.experimental import pallas as pl


def kernel(queries, keys, values, attn_mask):
    raise NotImplementedError("write your pallas kernel here")


# speedup vs baseline: 2.0350x; 2.0350x over previous
import math

import numpy as np
import jax
import jax.numpy as jnp
from jax.experimental import pallas as pl
from jax.experimental.pallas import tpu as pltpu

_L = 2048
_N1 = 64
_N2 = 32
_TOPK = int(math.log(_L))
_R = 32


def _fft_consts():
    n1 = np.arange(_N1)
    n2 = np.arange(_N2)
    w1 = np.exp(-2j * np.pi * np.outer(n1, n1) / _N1)
    w2 = np.exp(-2j * np.pi * np.outer(n2, n2) / _N2)
    tw = np.exp(-2j * np.pi * np.outer(n2, n1) / _L)
    w2c = np.exp(+2j * np.pi * np.outer(n2, n2) / _N2)
    w1c = np.exp(+2j * np.pi * np.outer(n1, n1) / _N1) / _L
    tw2 = np.exp(+2j * np.pi * np.outer(n1, n2) / _L)
    mask = np.ones((_N1, _N2), np.float32)
    mask[0, 0] = 0.0
    mask[0, (_L // 2) // _N1] = 0.0
    f32 = lambda a: jnp.asarray(a, jnp.float32)
    return (f32(w1.real), f32(w1.imag), f32(w2.real), f32(w2.imag),
            f32(tw.real), f32(tw.imag), f32(w2c.real), f32(w2c.imag),
            f32(w1c.real), f32(w1c.imag), f32(tw2.real), f32(tw2.imag),
            f32(mask))


def _body(q_ref, k_ref, v_ref,
          w1r_ref, w1i_ref, w2r_ref, w2i_ref, twr_ref, twi_ref,
          w2cr_ref, w2ci_ref, w1cr_ref, w1ci_ref, tw2r_ref, tw2i_ref,
          mask_ref, out_ref):
    w1r, w1i = w1r_ref[...], w1i_ref[...]
    w2r, w2i = w2r_ref[...], w2i_ref[...]
    twr, twi = twr_ref[None], twi_ref[None]
    w2cr, w2ci = w2cr_ref[...], w2ci_ref[...]
    w1cr, w1ci = w1cr_ref[...], w1ci_ref[...]
    tw2r, tw2i = tw2r_ref[None], tw2i_ref[None]
    mask = mask_ref[None]

    def mm(a, w):
        r, m, c = a.shape
        return jnp.dot(a.reshape(r * m, c), w,
                       precision=jax.lax.Precision.HIGHEST,
                       preferred_element_type=jnp.float32).reshape(r, m, -1)

    def swap(a):
        return jnp.swapaxes(a, 1, 2)

    def fft_fwd_real(x):
        ar = mm(x, w1r)
        ai = mm(x, w1i)
        br = ar * twr - ai * twi
        bi = ar * twi + ai * twr
        br, bi = swap(br), swap(bi)
        cr = mm(br, w2r) - mm(bi, w2i)
        ci = mm(br, w2i) + mm(bi, w2r)
        return cr, ci

    def ifft_real(xr, xi):
        zr = mm(xr, w2cr) - mm(xi, w2ci)
        zi = mm(xr, w2ci) + mm(xi, w2cr)
        yr = zr * tw2r - zi * tw2i
        yi = zr * tw2i + zi * tw2r
        yr, yi = swap(yr), swap(yi)
        return mm(yr, w1cr) - mm(yi, w1ci)

    q = q_ref[...]
    k = k_ref[...]
    v = v_ref[...]
    r = q.shape[0]

    qcr, qci = fft_fwd_real(q)
    kcr, kci = fft_fwd_real(k)
    pr = (qcr * kcr + qci * kci) * mask
    pi = (qci * kcr - qcr * kci) * mask
    corr = ifft_real(pr, pi)

    i2 = jax.lax.broadcasted_iota(jnp.int32, (r, _N2, _N1), 1)
    i3 = jax.lax.broadcasted_iota(jnp.int32, (r, _N2, _N1), 2)
    flat = i2 * _N1 + i3
    big = jnp.int32(_L)
    c = corr
    ws, js = [], []
    for _ in range(_TOPK):
        m = jnp.max(jnp.max(c, axis=2, keepdims=True), axis=1, keepdims=True)
        hit = c >= m
        j = jnp.min(jnp.min(jnp.where(hit, flat, big), axis=2, keepdims=True),
                    axis=1, keepdims=True)
        ws.append(m)
        js.append(j)
        c = jnp.where(flat == j, jnp.float32(-1e30), c)

    es = [jnp.exp(w - ws[0]) for w in ws]
    tot = es[0]
    for e in es[1:]:
        tot = tot + e
    inv_tot = 1.0 / tot

    g = jnp.zeros((r, _N2, _N1), jnp.float32)
    for e, j in zip(es, js):
        g = g + jnp.where(flat == j, e * inv_tot, 0.0)

    vcr, vci = fft_fwd_real(v)
    gcr, gci = fft_fwd_real(g)
    sr = vcr * gcr + vci * gci
    si = vci * gcr - vcr * gci
    out_ref[...] = ifft_real(sr, si)


def kernel(queries, keys, values, attn_mask):
    del attn_mask
    b, l, h, e = queries.shape
    rows = b * h * e

    def tile(x):
        x = jnp.transpose(x, (0, 2, 3, 1)).reshape(rows, _N1, _N2)
        return jnp.transpose(x, (0, 2, 1))

    qt, kt, vt = tile(queries), tile(keys), tile(values)
    consts = _fft_consts()

    cspec = [pl.BlockSpec(cst.shape, lambda i: (0,) * cst.ndim)
             for cst in consts]
    rspec = pl.BlockSpec((_R, _N2, _N1), lambda i: (i, 0, 0))

    out = pl.pallas_call(
        _body,
        grid=(rows // _R,),
        in_specs=[rspec, rspec, rspec] + cspec,
        out_specs=rspec,
        out_shape=jax.ShapeDtypeStruct((rows, _N2, _N1), jnp.float32),
    )(qt, kt, vt, *consts)

    out = jnp.transpose(out, (0, 2, 1)).reshape(b, h, e, l)
    return jnp.transpose(out, (0, 3, 1, 2))


# speedup vs baseline: 3.2561x; 1.6000x over previous
import math

import numpy as np
import jax
import jax.numpy as jnp
from jax.experimental import pallas as pl
from jax.experimental.pallas import tpu as pltpu

_L = 2048
_N1 = 64
_N2 = 32
_TOPK = int(math.log(_L))
_R = 32

_HI = jax.lax.Precision.HIGHEST
_LO = jax.lax.Precision.DEFAULT


def _fft_consts():
    n1 = np.arange(_N1)
    n2 = np.arange(_N2)
    w1 = np.exp(-2j * np.pi * np.outer(n1, n1) / _N1)
    w2 = np.exp(-2j * np.pi * np.outer(n2, n2) / _N2)
    tw = np.exp(-2j * np.pi * np.outer(n2, n1) / _L)
    w2c = np.exp(+2j * np.pi * np.outer(n2, n2) / _N2)
    w1c = np.exp(+2j * np.pi * np.outer(n1, n1) / _N1) / _L
    tw2 = np.exp(+2j * np.pi * np.outer(n1, n2) / _L)
    mask = np.ones((_N1, _N2), np.float32)
    mask[0, 0] = 0.0
    mask[0, (_L // 2) // _N1] = 0.0

    w1_ri = np.concatenate([w1.real, w1.imag], axis=1)
    w2_blk = np.block([[w2.real, w2.imag],
                       [-w2.imag, w2.real]])
    w2c_blk = np.block([[w2c.real, w2c.imag],
                        [-w2c.imag, w2c.real]])
    w1c_re = np.concatenate([w1c.real, -w1c.imag], axis=0)
    f32 = lambda a: jnp.asarray(a, jnp.float32)
    return (f32(w1_ri), f32(w2_blk), f32(w2c_blk), f32(w1c_re),
            f32(tw.real), f32(tw.imag), f32(tw2.real), f32(tw2.imag),
            f32(mask))


def _body(q_ref, k_ref, v_ref,
          w1ri_ref, w2blk_ref, w2cblk_ref, w1cre_ref,
          twr_ref, twi_ref, tw2r_ref, tw2i_ref, mask_ref, out_ref):
    w1ri = w1ri_ref[...]
    w2blk = w2blk_ref[...]
    w2cblk = w2cblk_ref[...]
    w1cre = w1cre_ref[...]
    twr, twi = twr_ref[None], twi_ref[None]
    tw2r, tw2i = tw2r_ref[None], tw2i_ref[None]
    mask = mask_ref[None]

    def mm(a, w, prec):
        r, m, c = a.shape
        return jnp.dot(a.reshape(r * m, c), w, precision=prec,
                       preferred_element_type=jnp.float32).reshape(r, m, -1)

    def swap(a):
        return jnp.swapaxes(a, 1, 2)

    def cat(a, b):
        return jnp.concatenate([a, b], axis=-1)

    def fft_fwd_real(x, prec):
        a = mm(x, w1ri, prec)
        ar, ai = a[:, :, :_N1], a[:, :, _N1:]
        br = ar * twr - ai * twi
        bi = ar * twi + ai * twr
        b = cat(swap(br), swap(bi))
        c = mm(b, w2blk, prec)
        return c[:, :, :_N2], c[:, :, _N2:]

    def ifft_real(xr, xi, prec):
        z = mm(cat(xr, xi), w2cblk, prec)
        zr, zi = z[:, :, :_N2], z[:, :, _N2:]
        yr = zr * tw2r - zi * tw2i
        yi = zr * tw2i + zi * tw2r
        y = cat(swap(yr), swap(yi))
        return mm(y, w1cre, prec)

    q = q_ref[...]
    k = k_ref[...]
    v = v_ref[...]
    r = q.shape[0]

    qcr, qci = fft_fwd_real(q, _HI)
    kcr, kci = fft_fwd_real(k, _HI)
    pr = (qcr * kcr + qci * kci) * mask
    pi = (qci * kcr - qcr * kci) * mask
    corr = ifft_real(pr, pi, _HI)

    i2 = jax.lax.broadcasted_iota(jnp.int32, (r, _N2, _N1), 1)
    i3 = jax.lax.broadcasted_iota(jnp.int32, (r, _N2, _N1), 2)
    flat = i2 * _N1 + i3
    big = jnp.int32(_L)
    c = corr
    ws, js = [], []
    for _ in range(_TOPK):
        m = jnp.max(jnp.max(c, axis=2, keepdims=True), axis=1, keepdims=True)
        hit = c >= m
        j = jnp.min(jnp.min(jnp.where(hit, flat, big), axis=2, keepdims=True),
                    axis=1, keepdims=True)
        ws.append(m)
        js.append(j)
        c = jnp.where(flat == j, jnp.float32(-1e30), c)

    es = [jnp.exp(w - ws[0]) for w in ws]
    tot = es[0]
    for e in es[1:]:
        tot = tot + e
    inv_tot = 1.0 / tot

    g = jnp.zeros((r, _N2, _N1), jnp.float32)
    for e, j in zip(es, js):
        g = g + jnp.where(flat == j, e * inv_tot, 0.0)

    vcr, vci = fft_fwd_real(v, _LO)
    gcr, gci = fft_fwd_real(g, _LO)
    sr = vcr * gcr + vci * gci
    si = vci * gcr - vcr * gci
    out_ref[...] = ifft_real(sr, si, _LO)


def kernel(queries, keys, values, attn_mask):
    del attn_mask
    b, l, h, e = queries.shape
    rows = b * h * e

    def tile(x):
        x = jnp.transpose(x, (0, 2, 3, 1)).reshape(rows, _N1, _N2)
        return jnp.transpose(x, (0, 2, 1))

    qt, kt, vt = tile(queries), tile(keys), tile(values)
    consts = _fft_consts()

    cspec = [pl.BlockSpec(cst.shape, lambda i: (0,) * cst.ndim)
             for cst in consts]
    rspec = pl.BlockSpec((_R, _N2, _N1), lambda i: (i, 0, 0))

    out = pl.pallas_call(
        _body,
        grid=(rows // _R,),
        in_specs=[rspec, rspec, rspec] + cspec,
        out_specs=rspec,
        out_shape=jax.ShapeDtypeStruct((rows, _N2, _N1), jnp.float32),
    )(qt, kt, vt, *consts)

    out = jnp.transpose(out, (0, 2, 1)).reshape(b, h, e, l)
    return jnp.transpose(out, (0, 3, 1, 2))


# speedup vs baseline: 3.3944x; 1.0425x over previous
import math

import numpy as np
import jax
import jax.numpy as jnp
from jax.experimental import pallas as pl
from jax.experimental.pallas import tpu as pltpu

_L = 2048
_N1 = 64
_N2 = 32
_TOPK = int(math.log(_L))
_R = 64

_HI = jax.lax.Precision.HIGHEST
_LO = jax.lax.Precision.DEFAULT


def _fft_consts():
    n1 = np.arange(_N1)
    n2 = np.arange(_N2)
    w1 = np.exp(-2j * np.pi * np.outer(n1, n1) / _N1)
    w2 = np.exp(-2j * np.pi * np.outer(n2, n2) / _N2)
    tw = np.exp(-2j * np.pi * np.outer(n2, n1) / _L)
    w2c = np.exp(+2j * np.pi * np.outer(n2, n2) / _N2)
    w1c = np.exp(+2j * np.pi * np.outer(n1, n1) / _N1) / _L
    tw2 = np.exp(+2j * np.pi * np.outer(n1, n2) / _L)
    mask = np.ones((_N1, _N2), np.float32)
    mask[0, 0] = 0.0
    mask[0, (_L // 2) // _N1] = 0.0

    w1_ri = np.concatenate([w1.real, w1.imag], axis=1)
    w2_blk = np.block([[w2.real, w2.imag],
                       [-w2.imag, w2.real]])
    w2c_blk = np.block([[w2c.real, w2c.imag],
                        [-w2c.imag, w2c.real]])
    w1c_re = np.concatenate([w1c.real, -w1c.imag], axis=0)
    f32 = lambda a: jnp.asarray(a, jnp.float32)
    return (f32(w1_ri), f32(w2_blk), f32(w2c_blk), f32(w1c_re),
            f32(tw.real), f32(tw.imag), f32(tw2.real), f32(tw2.imag),
            f32(mask))


def _body(q_ref, k_ref, v_ref,
          w1ri_ref, w2blk_ref, w2cblk_ref, w1cre_ref,
          twr_ref, twi_ref, tw2r_ref, tw2i_ref, mask_ref, out_ref):
    w1ri = w1ri_ref[...]
    w2blk = w2blk_ref[...]
    w2cblk = w2cblk_ref[...]
    w1cre = w1cre_ref[...]
    twr, twi = twr_ref[None], twi_ref[None]
    tw2r, tw2i = tw2r_ref[None], tw2i_ref[None]
    mask = mask_ref[None]

    def mm(a, w, prec):
        r, m, c = a.shape
        return jnp.dot(a.reshape(r * m, c), w, precision=prec,
                       preferred_element_type=jnp.float32).reshape(r, m, -1)

    def swap(a):
        return jnp.swapaxes(a, 1, 2)

    def cat(a, b):
        return jnp.concatenate([a, b], axis=-1)

    def fft_fwd_real(x, prec):
        a = mm(x, w1ri, prec)
        ar, ai = a[:, :, :_N1], a[:, :, _N1:]
        br = ar * twr - ai * twi
        bi = ar * twi + ai * twr
        b = cat(swap(br), swap(bi))
        c = mm(b, w2blk, prec)
        return c[:, :, :_N2], c[:, :, _N2:]

    def ifft_real(xr, xi, prec):
        z = mm(cat(xr, xi), w2cblk, prec)
        zr, zi = z[:, :, :_N2], z[:, :, _N2:]
        yr = zr * tw2r - zi * tw2i
        yi = zr * tw2i + zi * tw2r
        y = cat(swap(yr), swap(yi))
        return mm(y, w1cre, prec)

    q = q_ref[...]
    k = k_ref[...]
    v = v_ref[...]
    r = q.shape[0]

    qcr, qci = fft_fwd_real(q, _HI)
    kcr, kci = fft_fwd_real(k, _HI)
    pr = (qcr * kcr + qci * kci) * mask
    pi = (qci * kcr - qcr * kci) * mask
    corr = ifft_real(pr, pi, _HI)

    i2 = jax.lax.broadcasted_iota(jnp.int32, (r, _N2, _N1), 1)
    i3 = jax.lax.broadcasted_iota(jnp.int32, (r, _N2, _N1), 2)
    flat = i2 * _N1 + i3
    big = jnp.int32(_L)
    c = corr
    ws, js = [], []
    for _ in range(_TOPK):
        m = jnp.max(jnp.max(c, axis=2, keepdims=True), axis=1, keepdims=True)
        hit = c >= m
        j = jnp.min(jnp.min(jnp.where(hit, flat, big), axis=2, keepdims=True),
                    axis=1, keepdims=True)
        ws.append(m)
        js.append(j)
        c = jnp.where(flat == j, jnp.float32(-1e30), c)

    es = [jnp.exp(w - ws[0]) for w in ws]
    tot = es[0]
    for e in es[1:]:
        tot = tot + e
    inv_tot = 1.0 / tot

    g = jnp.zeros((r, _N2, _N1), jnp.float32)
    for e, j in zip(es, js):
        g = g + jnp.where(flat == j, e * inv_tot, 0.0)

    vcr, vci = fft_fwd_real(v, _LO)
    gcr, gci = fft_fwd_real(g, _LO)
    sr = vcr * gcr + vci * gci
    si = vci * gcr - vcr * gci
    out_ref[...] = ifft_real(sr, si, _LO)


def kernel(queries, keys, values, attn_mask):
    del attn_mask
    b, l, h, e = queries.shape
    rows = b * h * e

    def tile(x):
        x = jnp.transpose(x, (0, 2, 3, 1)).reshape(rows, _N1, _N2)
        return jnp.transpose(x, (0, 2, 1))

    qt, kt, vt = tile(queries), tile(keys), tile(values)
    consts = _fft_consts()

    cspec = [pl.BlockSpec(cst.shape, lambda i: (0,) * cst.ndim)
             for cst in consts]
    rspec = pl.BlockSpec((_R, _N2, _N1), lambda i: (i, 0, 0))

    out = pl.pallas_call(
        _body,
        grid=(rows // _R,),
        in_specs=[rspec, rspec, rspec] + cspec,
        out_specs=rspec,
        out_shape=jax.ShapeDtypeStruct((rows, _N2, _N1), jnp.float32),
    )(qt, kt, vt, *consts)

    out = jnp.transpose(out, (0, 2, 1)).reshape(b, h, e, l)
    return jnp.transpose(out, (0, 3, 1, 2))


# speedup vs baseline: 4.1540x; 1.2238x over previous
import math

import numpy as np
import jax
import jax.numpy as jnp
from jax.experimental import pallas as pl
from jax.experimental.pallas import tpu as pltpu

_L = 2048
_N1 = 64
_N2 = 32
_TOPK = int(math.log(_L))
_R = 64


def _split_bf16(a):
    hi = a.astype(np.float32).astype(jnp.bfloat16)
    lo = (a.astype(np.float32) - np.asarray(hi, np.float32)).astype(jnp.bfloat16)
    return jnp.asarray(hi), jnp.asarray(lo)


def _fft_consts():
    n1 = np.arange(_N1)
    n2 = np.arange(_N2)
    w1 = np.exp(-2j * np.pi * np.outer(n1, n1) / _N1)
    w2 = np.exp(-2j * np.pi * np.outer(n2, n2) / _N2)
    tw = np.exp(-2j * np.pi * np.outer(n2, n1) / _L)
    w2c = np.exp(+2j * np.pi * np.outer(n2, n2) / _N2)
    w1c = np.exp(+2j * np.pi * np.outer(n1, n1) / _N1) / _L
    tw2 = np.exp(+2j * np.pi * np.outer(n1, n2) / _L)
    mask = np.ones((_N1, _N2), np.float32)
    mask[0, 0] = 0.0
    mask[0, (_L // 2) // _N1] = 0.0

    w1_ri = np.concatenate([w1.real, w1.imag], axis=1)
    w2_blk = np.block([[w2.real, w2.imag],
                       [-w2.imag, w2.real]])
    w2c_blk = np.block([[w2c.real, w2c.imag],
                        [-w2c.imag, w2c.real]])
    w1c_re = np.concatenate([w1c.real, -w1c.imag], axis=0)
    f32 = lambda a: jnp.asarray(a, jnp.float32)
    return (_split_bf16(w1_ri) + _split_bf16(w2_blk) +
            _split_bf16(w2c_blk) + _split_bf16(w1c_re) +
            (f32(tw.real), f32(tw.imag), f32(tw2.real), f32(tw2.imag),
             f32(mask)))


def _body(q_ref, k_ref, v_ref,
          w1ri_h_ref, w1ri_l_ref, w2blk_h_ref, w2blk_l_ref,
          w2cblk_h_ref, w2cblk_l_ref, w1cre_h_ref, w1cre_l_ref,
          twr_ref, twi_ref, tw2r_ref, tw2i_ref, mask_ref, out_ref):
    w1ri = (w1ri_h_ref[...], w1ri_l_ref[...])
    w2blk = (w2blk_h_ref[...], w2blk_l_ref[...])
    w2cblk = (w2cblk_h_ref[...], w2cblk_l_ref[...])
    w1cre = (w1cre_h_ref[...], w1cre_l_ref[...])
    twr, twi = twr_ref[None], twi_ref[None]
    tw2r, tw2i = tw2r_ref[None], tw2i_ref[None]
    mask = mask_ref[None]

    def dot2d(a, w):
        return jnp.dot(a, w, preferred_element_type=jnp.float32)

    def mm_hi(a, w):
        r, m, c = a.shape
        a2 = a.reshape(r * m, c).astype(jnp.bfloat16)
        return dot2d(a2, w[0]).reshape(r, m, -1)

    def mm3(a, w):
        r, m, c = a.shape
        a2 = a.reshape(r * m, c)
        a_hi = a2.astype(jnp.bfloat16)
        a_lo = (a2 - a_hi.astype(jnp.float32)).astype(jnp.bfloat16)
        out = dot2d(a_hi, w[0]) + (dot2d(a_hi, w[1]) + dot2d(a_lo, w[0]))
        return out.reshape(r, m, -1)

    def swap(a):
        return jnp.swapaxes(a, 1, 2)

    def cat(a, b):
        return jnp.concatenate([a, b], axis=-1)

    def fft_fwd_real(x, mm):
        a = mm(x, w1ri)
        ar, ai = a[:, :, :_N1], a[:, :, _N1:]
        br = ar * twr - ai * twi
        bi = ar * twi + ai * twr
        b = cat(swap(br), swap(bi))
        c = mm(b, w2blk)
        return c[:, :, :_N2], c[:, :, _N2:]

    def ifft_real(xr, xi, mm):
        z = mm(cat(xr, xi), w2cblk)
        zr, zi = z[:, :, :_N2], z[:, :, _N2:]
        yr = zr * tw2r - zi * tw2i
        yi = zr * tw2i + zi * tw2r
        y = cat(swap(yr), swap(yi))
        return mm(y, w1cre)

    q = q_ref[...]
    k = k_ref[...]
    v = v_ref[...]
    r = q.shape[0]

    qcr, qci = fft_fwd_real(q, mm3)
    kcr, kci = fft_fwd_real(k, mm3)
    pr = (qcr * kcr + qci * kci) * mask
    pi = (qci * kcr - qcr * kci) * mask
    corr = ifft_real(pr, pi, mm3)

    i2 = jax.lax.broadcasted_iota(jnp.int32, (r, _N2, _N1), 1)
    i3 = jax.lax.broadcasted_iota(jnp.int32, (r, _N2, _N1), 2)
    flat = i2 * _N1 + i3
    big = jnp.int32(_L)
    c = corr
    ws, js = [], []
    for _ in range(_TOPK):
        m = jnp.max(jnp.max(c, axis=2, keepdims=True), axis=1, keepdims=True)
        hit = c >= m
        j = jnp.min(jnp.min(jnp.where(hit, flat, big), axis=2, keepdims=True),
                    axis=1, keepdims=True)
        ws.append(m)
        js.append(j)
        c = jnp.where(flat == j, jnp.float32(-1e30), c)

    es = [jnp.exp(w - ws[0]) for w in ws]
    tot = es[0]
    for e in es[1:]:
        tot = tot + e
    inv_tot = 1.0 / tot

    g = jnp.zeros((r, _N2, _N1), jnp.float32)
    for e, j in zip(es, js):
        g = g + jnp.where(flat == j, e * inv_tot, 0.0)

    vcr, vci = fft_fwd_real(v, mm_hi)
    gcr, gci = fft_fwd_real(g, mm_hi)
    sr = vcr * gcr + vci * gci
    si = vci * gcr - vcr * gci
    out_ref[...] = ifft_real(sr, si, mm_hi)


def kernel(queries, keys, values, attn_mask):
    del attn_mask
    b, l, h, e = queries.shape
    rows = b * h * e

    def tile(x):
        x = jnp.transpose(x, (0, 2, 3, 1)).reshape(rows, _N1, _N2)
        return jnp.transpose(x, (0, 2, 1))

    qt, kt, vt = tile(queries), tile(keys), tile(values)
    consts = _fft_consts()

    cspec = [pl.BlockSpec(cst.shape, lambda i: (0,) * cst.ndim)
             for cst in consts]
    rspec = pl.BlockSpec((_R, _N2, _N1), lambda i: (i, 0, 0))

    out = pl.pallas_call(
        _body,
        grid=(rows // _R,),
        in_specs=[rspec, rspec, rspec] + cspec,
        out_specs=rspec,
        out_shape=jax.ShapeDtypeStruct((rows, _N2, _N1), jnp.float32),
    )(qt, kt, vt, *consts)

    out = jnp.transpose(out, (0, 2, 1)).reshape(b, h, e, l)
    return jnp.transpose(out, (0, 3, 1, 2))
